# Initial kernel scaffold; baseline (speedup 1.0000x reference)
import jax
import jax.numpy as jnp
from jax.experimental import pallas as pl


def kernel(x, edge_index, W_rep0, W_rep2, W0, b0, W1, b1, W2, b2):
    raise NotImplementedError("write your pallas kernel here")


# speedup vs baseline: 18.5261x; 18.5261x over previous
import functools
import math

import jax
import jax.numpy as jnp
from jax import lax
from jax.experimental import pallas as pl
from jax.experimental.pallas import tpu as pltpu
from jax.experimental.pallas import tpu_sc as plsc

N = 10000
E = 320000
LO = -math.pi / 2 + 0.01
HI = math.pi / 2 - 0.01

NC, NS = 2, 16
NW = NC * NS
CHUNK = 128
RPT = 80
ROWS = RPT * NW
EPAD = ROWS * CHUNK
NPAD = 10112
JUNK = N
RT = NPAD // NS

_MESH = plsc.VectorSubcoreMesh(core_axis_name="c", subcore_axis_name="s")


def _deg_body(dst_hbm, init_hbm, out_hbm, dst_v, ones_v, stage_v, acc_sh):
    c = lax.axis_index("c")
    s = lax.axis_index("s")
    row0 = (c * NS + s) * RPT
    pltpu.sync_copy(dst_hbm.at[pl.ds(row0, RPT)], dst_v)
    for i in range(CHUNK // 16):
        ones_v[pl.ds(i * 16, 16)] = jnp.ones((16,), jnp.float32)
    pltpu.sync_copy(init_hbm.at[pl.ds(c * NPAD + s * RT, RT)], stage_v)
    pltpu.sync_copy(stage_v, acc_sh.at[pl.ds(s * RT, RT)])
    plsc.subcore_barrier()

    def step(j, carry):
        pltpu.sync_copy(ones_v, acc_sh.at[dst_v.at[j]], add=True)
        return carry

    lax.fori_loop(0, RPT, step, 0)
    plsc.subcore_barrier()
    pltpu.sync_copy(acc_sh.at[pl.ds(s * RT, RT)], stage_v)
    pltpu.sync_copy(stage_v, out_hbm.at[pl.ds(c * NPAD + s * RT, RT)])


def _deg_sc(dst_rows, init):
    return pl.kernel(
        _deg_body,
        out_type=jax.ShapeDtypeStruct((NC * NPAD,), jnp.float32),
        mesh=_MESH,
        scratch_types=[
            pltpu.VMEM((RPT, CHUNK), jnp.int32),
            pltpu.VMEM((CHUNK,), jnp.float32),
            pltpu.VMEM((RT,), jnp.float32),
            pltpu.VMEM_SHARED((NPAD,), jnp.float32),
        ],
        compiler_params=pltpu.CompilerParams(use_tc_tiling_on_sc=False),
    )(dst_rows, init)


def _agg_body(g_hbm, src_hbm, dst_hbm, zeros_hbm, out_hbm,
              src_v, dst_v, vals_v, stage_v, acc_sh, sem):
    c = lax.axis_index("c")
    s = lax.axis_index("s")
    row0 = (c * NS + s) * RPT
    pltpu.sync_copy(src_hbm.at[pl.ds(row0, RPT)], src_v)
    pltpu.sync_copy(dst_hbm.at[pl.ds(row0, RPT)], dst_v)
    pltpu.sync_copy(zeros_hbm.at[pl.ds(s * RT, RT)], stage_v)
    pltpu.sync_copy(stage_v, acc_sh.at[pl.ds(s * RT, RT)])
    plsc.subcore_barrier()

    def step(j, carry):
        pltpu.async_copy(g_hbm.at[src_v.at[j]], vals_v, sem).wait()
        pltpu.sync_copy(vals_v, acc_sh.at[dst_v.at[j]], add=True)
        return carry

    lax.fori_loop(0, RPT, step, 0)
    plsc.subcore_barrier()
    pltpu.sync_copy(acc_sh.at[pl.ds(s * RT, RT)], stage_v)
    pltpu.sync_copy(stage_v, out_hbm.at[c, pl.ds(s * RT, RT)])


def _seg_sum_sc(g, src_rows, dst_rows, feat, zeros):
    return pl.kernel(
        _agg_body,
        out_type=jax.ShapeDtypeStruct((NC, NPAD, feat), jnp.float32),
        mesh=_MESH,
        scratch_types=[
            pltpu.VMEM((RPT, CHUNK), jnp.int32),
            pltpu.VMEM((RPT, CHUNK), jnp.int32),
            pltpu.VMEM((CHUNK, feat), jnp.float32),
            pltpu.VMEM((RT, feat), jnp.float32),
            pltpu.VMEM_SHARED((NPAD, feat), jnp.float32),
            pltpu.SemaphoreType.DMA,
        ],
        compiler_params=pltpu.CompilerParams(use_tc_tiling_on_sc=False),
    )(g, src_rows, dst_rows, zeros)


def _a0_body(x_ref, w0pt_ref, w0t_ref, dinv_ref, tanoff_ref, g_ref):
    xx = x_ref[...]
    off = jnp.clip(jnp.dot(xx, w0pt_ref[...], preferred_element_type=jnp.float32),
                   LO, HI)
    tanoff_ref[...] = jnp.tan(off)
    g_ref[...] = dinv_ref[...] * jnp.dot(xx, w0t_ref[...],
                                         preferred_element_type=jnp.float32)


def _a1_body(tanoff_ref, a0_ref, a1_ref, g_ref, dinv_ref, b_ref, wt_ref,
             tanoff_o, g_o):
    dinv = dinv_ref[...]
    h = jnp.arctan2(tanoff_ref[...]
                    + dinv * (a0_ref[...] + a1_ref[...] + g_ref[...])
                    + b_ref[...], 1.0)
    tanoff_o[...] = jnp.tan(jnp.clip(h, LO, HI))
    g_o[...] = dinv * jnp.dot(h, wt_ref[...], preferred_element_type=jnp.float32)


def _a2_body(tanoff_ref, a0_ref, a1_ref, g_ref, dinv_ref, b_ref, wpt_ref, wt_ref,
             tanoff_o, g_o):
    dinv = dinv_ref[...]
    h = jnp.arctan2(tanoff_ref[...]
                    + dinv * (a0_ref[...] + a1_ref[...] + g_ref[...])
                    + b_ref[...], 1.0)
    off = jnp.dot(h, wpt_ref[...], preferred_element_type=jnp.float32)
    tanoff_o[...] = jnp.tan(jnp.clip(off, LO, HI))
    g_o[...] = dinv * jnp.dot(h, wt_ref[...], preferred_element_type=jnp.float32)


def _b2_body(tanoff_ref, a0_ref, a1_ref, g_ref, dinv_ref, b_ref, out_ref):
    h = jnp.arctan2(tanoff_ref[...]
                    + dinv_ref[...] * (a0_ref[...] + a1_ref[...] + g_ref[...])
                    + b_ref[...], 1.0)
    m = jnp.max(h, axis=1, keepdims=True)
    lse = m + jnp.log(jnp.sum(jnp.exp(h - m), axis=1, keepdims=True))
    out_ref[...] = h - lse


def _sds(shape):
    return jax.ShapeDtypeStruct(shape, jnp.float32)


def _proj_simplex(v, radius=1.0):
    n_feat = v.shape[1]
    u = jnp.sort(v, axis=1)
    cssv = jnp.cumsum(u, axis=1) - radius
    ind = jnp.arange(1, n_feat + 1)
    cond = u - cssv / ind.astype(v.dtype) > 0
    rho = jnp.max(jnp.where(cond, ind, 0), axis=1)
    theta = jnp.take_along_axis(cssv, (rho - 1)[:, None], axis=1)[:, 0]
    theta = theta / rho.astype(v.dtype)
    return jax.nn.relu(v - theta[:, None])


def kernel(x, edge_index, W_rep0, W_rep2, W0, b0, W1, b1, W2, b2):
    f32 = jnp.float32
    src = edge_index[0]
    dst = edge_index[1]
    pad = EPAD - E
    src_rows = jnp.concatenate(
        [src, jnp.zeros((pad,), jnp.int32)]).reshape(ROWS, CHUNK)
    dst_rows = jnp.concatenate(
        [dst, jnp.full((pad,), JUNK, jnp.int32)]).reshape(ROWS, CHUNK)

    deg_init = jnp.concatenate(
        [jnp.ones((NPAD,), f32), jnp.zeros((NPAD,), f32)])
    degp = _deg_sc(dst_rows, deg_init)
    deg = degp[:N] + degp[NPAD:NPAD + N]
    dinv = (deg ** -0.5).reshape(N, 1)

    W0p = _proj_simplex(W_rep0)
    W2p = _proj_simplex(W_rep2)

    zeros16 = jnp.zeros((NPAD, 16), f32)
    zeros40 = jnp.zeros((NPAD, 40), f32)

    tanoff0, g0 = pl.pallas_call(
        _a0_body, out_shape=[_sds((N, 16)), _sds((N, 16))],
    )(x, W0p.T, W0.T, dinv)

    agg0 = _seg_sum_sc(g0, src_rows, dst_rows, 16, zeros16)

    tanoff1, g1 = pl.pallas_call(
        _a1_body, out_shape=[_sds((N, 16)), _sds((N, 16))],
    )(tanoff0, agg0[0, :N], agg0[1, :N], g0, dinv, b0.reshape(1, 16), W1.T)

    agg1 = _seg_sum_sc(g1, src_rows, dst_rows, 16, zeros16)

    tanoff2, g2 = pl.pallas_call(
        _a2_body, out_shape=[_sds((N, 40)), _sds((N, 40))],
    )(tanoff1, agg1[0, :N], agg1[1, :N], g1, dinv, b1.reshape(1, 16),
      W2p.T, W2.T)

    agg2 = _seg_sum_sc(g2, src_rows, dst_rows, 40, zeros40)

    out = pl.pallas_call(
        _b2_body, out_shape=_sds((N, 40)),
    )(tanoff2, agg2[0, :N], agg2[1, :N], g2, dinv, b2.reshape(1, 40))
    return out


# speedup vs baseline: 21.4724x; 1.1590x over previous
import functools
import math

import jax
import jax.numpy as jnp
from jax import lax
from jax.experimental import pallas as pl
from jax.experimental.pallas import tpu as pltpu
from jax.experimental.pallas import tpu_sc as plsc

N = 10000
E = 320000
LO = -math.pi / 2 + 0.01
HI = math.pi / 2 - 0.01

NC, NS = 2, 16
NW = NC * NS
CHUNK = 128
RPT = 80
ROWS = RPT * NW
EPAD = ROWS * CHUNK
NPAD = 10112
JUNK = N
RT = NPAD // NS

_MESH = plsc.VectorSubcoreMesh(core_axis_name="c", subcore_axis_name="s")


def _deg_body(dst_hbm, init_hbm, out_hbm, dst_v, ones_v, stage_v, acc_sh):
    c = lax.axis_index("c")
    s = lax.axis_index("s")
    row0 = (c * NS + s) * RPT
    pltpu.sync_copy(dst_hbm.at[pl.ds(row0, RPT)], dst_v)
    for i in range(CHUNK // 16):
        ones_v[pl.ds(i * 16, 16)] = jnp.ones((16,), jnp.float32)
    pltpu.sync_copy(init_hbm.at[pl.ds(c * NPAD + s * RT, RT)], stage_v)
    pltpu.sync_copy(stage_v, acc_sh.at[pl.ds(s * RT, RT)])
    plsc.subcore_barrier()

    def step(j, carry):
        pltpu.sync_copy(ones_v, acc_sh.at[dst_v.at[j]], add=True)
        return carry

    lax.fori_loop(0, RPT, step, 0)
    plsc.subcore_barrier()
    pltpu.sync_copy(acc_sh.at[pl.ds(s * RT, RT)], stage_v)
    pltpu.sync_copy(stage_v, out_hbm.at[pl.ds(c * NPAD + s * RT, RT)])


def _deg_sc(dst_rows, init):
    return pl.kernel(
        _deg_body,
        out_type=jax.ShapeDtypeStruct((NC * NPAD,), jnp.float32),
        mesh=_MESH,
        scratch_types=[
            pltpu.VMEM((RPT, CHUNK), jnp.int32),
            pltpu.VMEM((CHUNK,), jnp.float32),
            pltpu.VMEM((RT,), jnp.float32),
            pltpu.VMEM_SHARED((NPAD,), jnp.float32),
        ],
        compiler_params=pltpu.CompilerParams(use_tc_tiling_on_sc=False),
    )(dst_rows, init)


KG = 4
NG = RPT // KG


def _agg_body(g_hbm, src_hbm, dst_hbm, zeros_hbm, out_hbm,
              src_v, dst_v, vals_v, stage_v, acc_sh,
              gs_a, gs_b, ss_a, ss_b):
    c = lax.axis_index("c")
    s = lax.axis_index("s")
    row0 = (c * NS + s) * RPT
    pltpu.sync_copy(src_hbm.at[pl.ds(row0, RPT)], src_v)
    pltpu.sync_copy(dst_hbm.at[pl.ds(row0, RPT)], dst_v)
    pltpu.sync_copy(zeros_hbm.at[pl.ds(s * RT, RT)], stage_v)
    pltpu.sync_copy(stage_v, acc_sh.at[pl.ds(s * RT, RT)])
    plsc.subcore_barrier()

    gsems = (gs_a, gs_b)
    ssems = (ss_a, ss_b)
    last = RPT - 1

    def gath(phase, b, j):
        pltpu.async_copy(g_hbm.at[src_v.at[j]], vals_v.at[phase, b],
                         gsems[phase])

    def drain_g(phase, b):
        pltpu.make_async_copy(g_hbm.at[pl.ds(0, CHUNK)], vals_v.at[phase, b],
                              gsems[phase]).wait()

    def scat(phase, b, j):
        pltpu.async_copy(vals_v.at[phase, b], acc_sh.at[dst_v.at[j]],
                         ssems[phase], add=True)

    def drain_s(phase, b):
        pltpu.make_async_copy(g_hbm.at[pl.ds(0, CHUNK)], vals_v.at[phase, b],
                              ssems[phase]).wait()

    for b in range(KG):
        gath(0, b, b)
    for b in range(KG):
        gath(1, b, KG + b)

    def body(i, carry):
        base_a = (2 * i) * KG
        base_b = base_a + KG
        for b in range(KG):
            drain_g(0, b)
        for b in range(KG):
            scat(0, b, base_a + b)
        for b in range(KG):
            drain_g(1, b)
        for b in range(KG):
            scat(1, b, base_b + b)
        for b in range(KG):
            drain_s(0, b)
        for b in range(KG):
            gath(0, b, jnp.minimum(base_a + 2 * KG + b, last))
        for b in range(KG):
            drain_s(1, b)
        for b in range(KG):
            gath(1, b, jnp.minimum(base_b + 2 * KG + b, last))
        return carry

    lax.fori_loop(0, NG // 2, body, 0)
    for phase in (0, 1):
        for b in range(KG):
            drain_g(phase, b)
    plsc.subcore_barrier()
    pltpu.sync_copy(acc_sh.at[pl.ds(s * RT, RT)], stage_v)
    pltpu.sync_copy(stage_v, out_hbm.at[c, pl.ds(s * RT, RT)])


def _seg_sum_sc(g, src_rows, dst_rows, feat, zeros):
    return pl.kernel(
        _agg_body,
        out_type=jax.ShapeDtypeStruct((NC, NPAD, feat), jnp.float32),
        mesh=_MESH,
        scratch_types=[
            pltpu.VMEM((RPT, CHUNK), jnp.int32),
            pltpu.VMEM((RPT, CHUNK), jnp.int32),
            pltpu.VMEM((2, KG, CHUNK, feat), jnp.float32),
            pltpu.VMEM((RT, feat), jnp.float32),
            pltpu.VMEM_SHARED((NPAD, feat), jnp.float32),
            pltpu.SemaphoreType.DMA,
            pltpu.SemaphoreType.DMA,
            pltpu.SemaphoreType.DMA,
            pltpu.SemaphoreType.DMA,
        ],
        compiler_params=pltpu.CompilerParams(use_tc_tiling_on_sc=False),
    )(g, src_rows, dst_rows, zeros)


def _a0_body(x_ref, w0pt_ref, w0t_ref, dinv_ref, tanoff_ref, g_ref):
    xx = x_ref[...]
    off = jnp.clip(jnp.dot(xx, w0pt_ref[...], preferred_element_type=jnp.float32),
                   LO, HI)
    tanoff_ref[...] = jnp.tan(off)
    g_ref[...] = dinv_ref[...] * jnp.dot(xx, w0t_ref[...],
                                         preferred_element_type=jnp.float32)


def _a1_body(tanoff_ref, a0_ref, a1_ref, g_ref, dinv_ref, b_ref, wt_ref,
             tanoff_o, g_o):
    dinv = dinv_ref[...]
    h = jnp.arctan2(tanoff_ref[...]
                    + dinv * (a0_ref[...] + a1_ref[...] + g_ref[...])
                    + b_ref[...], 1.0)
    tanoff_o[...] = jnp.tan(jnp.clip(h, LO, HI))
    g_o[...] = dinv * jnp.dot(h, wt_ref[...], preferred_element_type=jnp.float32)


def _a2_body(tanoff_ref, a0_ref, a1_ref, g_ref, dinv_ref, b_ref, wpt_ref, wt_ref,
             tanoff_o, g_o):
    dinv = dinv_ref[...]
    h = jnp.arctan2(tanoff_ref[...]
                    + dinv * (a0_ref[...] + a1_ref[...] + g_ref[...])
                    + b_ref[...], 1.0)
    off = jnp.dot(h, wpt_ref[...], preferred_element_type=jnp.float32)
    tanoff_o[...] = jnp.tan(jnp.clip(off, LO, HI))
    g_o[...] = dinv * jnp.dot(h, wt_ref[...], preferred_element_type=jnp.float32)


def _b2_body(tanoff_ref, a0_ref, a1_ref, g_ref, dinv_ref, b_ref, out_ref):
    h = jnp.arctan2(tanoff_ref[...]
                    + dinv_ref[...] * (a0_ref[...] + a1_ref[...] + g_ref[...])
                    + b_ref[...], 1.0)
    m = jnp.max(h, axis=1, keepdims=True)
    lse = m + jnp.log(jnp.sum(jnp.exp(h - m), axis=1, keepdims=True))
    out_ref[...] = h - lse


def _sds(shape):
    return jax.ShapeDtypeStruct(shape, jnp.float32)


def _proj_simplex(v, radius=1.0):
    n_feat = v.shape[1]
    u = jnp.sort(v, axis=1)
    cssv = jnp.cumsum(u, axis=1) - radius
    ind = jnp.arange(1, n_feat + 1)
    cond = u - cssv / ind.astype(v.dtype) > 0
    rho = jnp.max(jnp.where(cond, ind, 0), axis=1)
    theta = jnp.take_along_axis(cssv, (rho - 1)[:, None], axis=1)[:, 0]
    theta = theta / rho.astype(v.dtype)
    return jax.nn.relu(v - theta[:, None])


def kernel(x, edge_index, W_rep0, W_rep2, W0, b0, W1, b1, W2, b2):
    f32 = jnp.float32
    src = edge_index[0]
    dst = edge_index[1]
    pad = EPAD - E
    src_rows = jnp.concatenate(
        [src, jnp.zeros((pad,), jnp.int32)]).reshape(ROWS, CHUNK)
    dst_rows = jnp.concatenate(
        [dst, jnp.full((pad,), JUNK, jnp.int32)]).reshape(ROWS, CHUNK)

    deg_init = jnp.concatenate(
        [jnp.ones((NPAD,), f32), jnp.zeros((NPAD,), f32)])
    degp = _deg_sc(dst_rows, deg_init)
    deg = degp[:N] + degp[NPAD:NPAD + N]
    dinv = (deg ** -0.5).reshape(N, 1)

    W0p = _proj_simplex(W_rep0)
    W2p = _proj_simplex(W_rep2)

    zeros16 = jnp.zeros((NPAD, 16), f32)
    zeros40 = jnp.zeros((NPAD, 40), f32)

    tanoff0, g0 = pl.pallas_call(
        _a0_body, out_shape=[_sds((N, 16)), _sds((N, 16))],
    )(x, W0p.T, W0.T, dinv)

    agg0 = _seg_sum_sc(g0, src_rows, dst_rows, 16, zeros16)

    tanoff1, g1 = pl.pallas_call(
        _a1_body, out_shape=[_sds((N, 16)), _sds((N, 16))],
    )(tanoff0, agg0[0, :N], agg0[1, :N], g0, dinv, b0.reshape(1, 16), W1.T)

    agg1 = _seg_sum_sc(g1, src_rows, dst_rows, 16, zeros16)

    tanoff2, g2 = pl.pallas_call(
        _a2_body, out_shape=[_sds((N, 40)), _sds((N, 40))],
    )(tanoff1, agg1[0, :N], agg1[1, :N], g1, dinv, b1.reshape(1, 16),
      W2p.T, W2.T)

    agg2 = _seg_sum_sc(g2, src_rows, dst_rows, 40, zeros40)

    out = pl.pallas_call(
        _b2_body, out_shape=_sds((N, 40)),
    )(tanoff2, agg2[0, :N], agg2[1, :N], g2, dinv, b2.reshape(1, 40))
    return out


# speedup vs baseline: 24.6362x; 1.1473x over previous
import functools
import math

import jax
import jax.numpy as jnp
from jax import lax
from jax.experimental import pallas as pl
from jax.experimental.pallas import tpu as pltpu
from jax.experimental.pallas import tpu_sc as plsc

N = 10000
E = 320000
LO = -math.pi / 2 + 0.01
HI = math.pi / 2 - 0.01

NC, NS = 2, 16
NW = NC * NS
CHUNK = 128
RPT = 80
ROWS = RPT * NW
EPAD = ROWS * CHUNK
NPAD = 10112
JUNK = N
RT = NPAD // NS

_MESH = plsc.VectorSubcoreMesh(core_axis_name="c", subcore_axis_name="s")


def _deg_body(dst_hbm, init_hbm, out_hbm, dst_v, ones_v, stage_v, acc_sh):
    c = lax.axis_index("c")
    s = lax.axis_index("s")
    row0 = (c * NS + s) * RPT
    pltpu.sync_copy(dst_hbm.at[pl.ds(row0, RPT)], dst_v)
    for i in range(CHUNK // 16):
        ones_v[pl.ds(i * 16, 16)] = jnp.ones((16,), jnp.float32)
    pltpu.sync_copy(init_hbm.at[pl.ds(c * NPAD + s * RT, RT)], stage_v)
    pltpu.sync_copy(stage_v, acc_sh.at[pl.ds(s * RT, RT)])
    plsc.subcore_barrier()

    def step(j, carry):
        pltpu.sync_copy(ones_v, acc_sh.at[dst_v.at[j]], add=True)
        return carry

    lax.fori_loop(0, RPT, step, 0)
    plsc.subcore_barrier()
    pltpu.sync_copy(acc_sh.at[pl.ds(s * RT, RT)], stage_v)
    pltpu.sync_copy(stage_v, out_hbm.at[pl.ds(c * NPAD + s * RT, RT)])


def _deg_sc(dst_rows, init):
    return pl.kernel(
        _deg_body,
        out_type=jax.ShapeDtypeStruct((NC * NPAD,), jnp.float32),
        mesh=_MESH,
        scratch_types=[
            pltpu.VMEM((RPT, CHUNK), jnp.int32),
            pltpu.VMEM((CHUNK,), jnp.float32),
            pltpu.VMEM((RT,), jnp.float32),
            pltpu.VMEM_SHARED((NPAD,), jnp.float32),
        ],
        compiler_params=pltpu.CompilerParams(use_tc_tiling_on_sc=False),
    )(dst_rows, init)


KG = 4
NG = RPT // KG


def _make_agg_body(stage_table):
    def _agg_body(g_hbm, src_hbm, dst_hbm, zeros_hbm, out_hbm,
                  src_v, dst_v, vals_v, stage_v, acc_sh, g_sh,
                  gs_a, gs_b, ss_a, ss_b):
        c = lax.axis_index("c")
        s = lax.axis_index("s")
        row0 = (c * NS + s) * RPT
        pltpu.sync_copy(src_hbm.at[pl.ds(row0, RPT)], src_v)
        pltpu.sync_copy(dst_hbm.at[pl.ds(row0, RPT)], dst_v)
        if stage_table:
            pltpu.sync_copy(g_hbm.at[pl.ds(s * RT, RT)], stage_v)
            pltpu.sync_copy(stage_v, g_sh.at[pl.ds(s * RT, RT)])
        pltpu.sync_copy(zeros_hbm.at[pl.ds(s * RT, RT)], stage_v)
        pltpu.sync_copy(stage_v, acc_sh.at[pl.ds(s * RT, RT)])
        plsc.subcore_barrier()

        gtab = g_sh if stage_table else g_hbm
        gsems = (gs_a, gs_b)
        ssems = (ss_a, ss_b)
        last = RPT - 1

        def gath(phase, b, j):
            pltpu.async_copy(gtab.at[src_v.at[j]], vals_v.at[phase, b],
                             gsems[phase])

        def drain_g(phase, b):
            pltpu.make_async_copy(g_hbm.at[pl.ds(0, CHUNK)],
                                  vals_v.at[phase, b], gsems[phase]).wait()

        def scat(phase, b, j):
            pltpu.async_copy(vals_v.at[phase, b], acc_sh.at[dst_v.at[j]],
                             ssems[phase], add=True)

        def drain_s(phase, b):
            pltpu.make_async_copy(g_hbm.at[pl.ds(0, CHUNK)],
                                  vals_v.at[phase, b], ssems[phase]).wait()

        for b in range(KG):
            gath(0, b, b)
        for b in range(KG):
            gath(1, b, KG + b)

        def body(i, carry):
            base_a = (2 * i) * KG
            base_b = base_a + KG
            for b in range(KG):
                drain_g(0, b)
            for b in range(KG):
                scat(0, b, base_a + b)
            for b in range(KG):
                drain_g(1, b)
            for b in range(KG):
                scat(1, b, base_b + b)
            for b in range(KG):
                drain_s(0, b)
            for b in range(KG):
                gath(0, b, jnp.minimum(base_a + 2 * KG + b, last))
            for b in range(KG):
                drain_s(1, b)
            for b in range(KG):
                gath(1, b, jnp.minimum(base_b + 2 * KG + b, last))
            return carry

        lax.fori_loop(0, NG // 2, body, 0)
        for phase in (0, 1):
            for b in range(KG):
                drain_g(phase, b)
        plsc.subcore_barrier()
        pltpu.sync_copy(acc_sh.at[pl.ds(s * RT, RT)], stage_v)
        pltpu.sync_copy(stage_v, out_hbm.at[c, pl.ds(s * RT, RT)])

    return _agg_body


def _seg_sum_sc(g, src_rows, dst_rows, feat, zeros, stage_table):
    g = jnp.pad(g, ((0, NPAD - N), (0, 0)))
    gsh_rows = NPAD if stage_table else 8
    return pl.kernel(
        _make_agg_body(stage_table),
        out_type=jax.ShapeDtypeStruct((NC, NPAD, feat), jnp.float32),
        mesh=_MESH,
        scratch_types=[
            pltpu.VMEM((RPT, CHUNK), jnp.int32),
            pltpu.VMEM((RPT, CHUNK), jnp.int32),
            pltpu.VMEM((2, KG, CHUNK, feat), jnp.float32),
            pltpu.VMEM((RT, feat), jnp.float32),
            pltpu.VMEM_SHARED((NPAD, feat), jnp.float32),
            pltpu.VMEM_SHARED((gsh_rows, feat), jnp.float32),
            pltpu.SemaphoreType.DMA,
            pltpu.SemaphoreType.DMA,
            pltpu.SemaphoreType.DMA,
            pltpu.SemaphoreType.DMA,
        ],
        compiler_params=pltpu.CompilerParams(use_tc_tiling_on_sc=False),
    )(g, src_rows, dst_rows, zeros)


def _a0_body(x_ref, w0pt_ref, w0t_ref, dinv_ref, tanoff_ref, g_ref):
    xx = x_ref[...]
    off = jnp.clip(jnp.dot(xx, w0pt_ref[...], preferred_element_type=jnp.float32),
                   LO, HI)
    tanoff_ref[...] = jnp.tan(off)
    g_ref[...] = dinv_ref[...] * jnp.dot(xx, w0t_ref[...],
                                         preferred_element_type=jnp.float32)


def _a1_body(tanoff_ref, a0_ref, a1_ref, g_ref, dinv_ref, b_ref, wt_ref,
             tanoff_o, g_o):
    dinv = dinv_ref[...]
    h = jnp.arctan2(tanoff_ref[...]
                    + dinv * (a0_ref[...] + a1_ref[...] + g_ref[...])
                    + b_ref[...], 1.0)
    tanoff_o[...] = jnp.tan(jnp.clip(h, LO, HI))
    g_o[...] = dinv * jnp.dot(h, wt_ref[...], preferred_element_type=jnp.float32)


def _a2_body(tanoff_ref, a0_ref, a1_ref, g_ref, dinv_ref, b_ref, wpt_ref, wt_ref,
             tanoff_o, g_o):
    dinv = dinv_ref[...]
    h = jnp.arctan2(tanoff_ref[...]
                    + dinv * (a0_ref[...] + a1_ref[...] + g_ref[...])
                    + b_ref[...], 1.0)
    off = jnp.dot(h, wpt_ref[...], preferred_element_type=jnp.float32)
    tanoff_o[...] = jnp.tan(jnp.clip(off, LO, HI))
    g_o[...] = dinv * jnp.dot(h, wt_ref[...], preferred_element_type=jnp.float32)


def _b2_body(tanoff_ref, a0_ref, a1_ref, g_ref, dinv_ref, b_ref, out_ref):
    h = jnp.arctan2(tanoff_ref[...]
                    + dinv_ref[...] * (a0_ref[...] + a1_ref[...] + g_ref[...])
                    + b_ref[...], 1.0)
    m = jnp.max(h, axis=1, keepdims=True)
    lse = m + jnp.log(jnp.sum(jnp.exp(h - m), axis=1, keepdims=True))
    out_ref[...] = h - lse


def _sds(shape):
    return jax.ShapeDtypeStruct(shape, jnp.float32)


def _proj_simplex(v, radius=1.0):
    n_feat = v.shape[1]
    u = jnp.sort(v, axis=1)
    cssv = jnp.cumsum(u, axis=1) - radius
    ind = jnp.arange(1, n_feat + 1)
    cond = u - cssv / ind.astype(v.dtype) > 0
    rho = jnp.max(jnp.where(cond, ind, 0), axis=1)
    theta = jnp.take_along_axis(cssv, (rho - 1)[:, None], axis=1)[:, 0]
    theta = theta / rho.astype(v.dtype)
    return jax.nn.relu(v - theta[:, None])


def kernel(x, edge_index, W_rep0, W_rep2, W0, b0, W1, b1, W2, b2):
    f32 = jnp.float32
    src = edge_index[0]
    dst = edge_index[1]
    pad = EPAD - E
    src_rows = jnp.concatenate(
        [src, jnp.zeros((pad,), jnp.int32)]).reshape(ROWS, CHUNK)
    dst_rows = jnp.concatenate(
        [dst, jnp.full((pad,), JUNK, jnp.int32)]).reshape(ROWS, CHUNK)

    deg_init = jnp.concatenate(
        [jnp.ones((NPAD,), f32), jnp.zeros((NPAD,), f32)])
    degp = _deg_sc(dst_rows, deg_init)
    deg = degp[:N] + degp[NPAD:NPAD + N]
    dinv = (deg ** -0.5).reshape(N, 1)

    W0p = _proj_simplex(W_rep0)
    W2p = _proj_simplex(W_rep2)

    zeros16 = jnp.zeros((NPAD, 16), f32)
    zeros40 = jnp.zeros((NPAD, 40), f32)

    tanoff0, g0 = pl.pallas_call(
        _a0_body, out_shape=[_sds((N, 16)), _sds((N, 16))],
    )(x, W0p.T, W0.T, dinv)

    agg0 = _seg_sum_sc(g0, src_rows, dst_rows, 16, zeros16, True)

    tanoff1, g1 = pl.pallas_call(
        _a1_body, out_shape=[_sds((N, 16)), _sds((N, 16))],
    )(tanoff0, agg0[0, :N], agg0[1, :N], g0, dinv, b0.reshape(1, 16), W1.T)

    agg1 = _seg_sum_sc(g1, src_rows, dst_rows, 16, zeros16, True)

    tanoff2, g2 = pl.pallas_call(
        _a2_body, out_shape=[_sds((N, 40)), _sds((N, 40))],
    )(tanoff1, agg1[0, :N], agg1[1, :N], g1, dinv, b1.reshape(1, 16),
      W2p.T, W2.T)

    agg2 = _seg_sum_sc(g2, src_rows, dst_rows, 40, zeros40, False)

    out = pl.pallas_call(
        _b2_body, out_shape=_sds((N, 40)),
    )(tanoff2, agg2[0, :N], agg2[1, :N], g2, dinv, b2.reshape(1, 40))
    return out


# speedup vs baseline: 32.2031x; 1.3071x over previous
import functools
import math

import jax
import jax.numpy as jnp
from jax import lax
from jax.experimental import pallas as pl
from jax.experimental.pallas import tpu as pltpu
from jax.experimental.pallas import tpu_sc as plsc

N = 10000
E = 320000
LO = -math.pi / 2 + 0.01
HI = math.pi / 2 - 0.01
THI = math.tan(HI)
TLO = -THI

NC, NS = 2, 16
NW = NC * NS
CHUNK = 128
RPT = 80
ROWS = RPT * NW
EPAD = ROWS * CHUNK
NPAD = 10112
JUNK = N
RT = NPAD // NS

_MESH = plsc.VectorSubcoreMesh(core_axis_name="c", subcore_axis_name="s")


def _deg_body(dst_hbm, init_hbm, out_hbm, dst_v, ones_v, stage_v, acc_sh):
    c = lax.axis_index("c")
    s = lax.axis_index("s")
    row0 = (c * NS + s) * RPT
    pltpu.sync_copy(dst_hbm.at[pl.ds(row0, RPT)], dst_v)
    for i in range(CHUNK // 16):
        ones_v[pl.ds(i * 16, 16)] = jnp.ones((16,), jnp.float32)
    pltpu.sync_copy(init_hbm.at[pl.ds(c * NPAD + s * RT, RT)], stage_v)
    pltpu.sync_copy(stage_v, acc_sh.at[pl.ds(s * RT, RT)])
    plsc.subcore_barrier()

    def step(j, carry):
        pltpu.sync_copy(ones_v, acc_sh.at[dst_v.at[j]], add=True)
        return carry

    lax.fori_loop(0, RPT, step, 0)
    plsc.subcore_barrier()
    pltpu.sync_copy(acc_sh.at[pl.ds(s * RT, RT)], stage_v)
    pltpu.sync_copy(stage_v, out_hbm.at[pl.ds(c * NPAD + s * RT, RT)])


def _deg_sc(dst_rows, init):
    return pl.kernel(
        _deg_body,
        out_type=jax.ShapeDtypeStruct((NC * NPAD,), jnp.float32),
        mesh=_MESH,
        scratch_types=[
            pltpu.VMEM((RPT, CHUNK), jnp.int32),
            pltpu.VMEM((CHUNK,), jnp.float32),
            pltpu.VMEM((RT,), jnp.float32),
            pltpu.VMEM_SHARED((NPAD,), jnp.float32),
        ],
        compiler_params=pltpu.CompilerParams(use_tc_tiling_on_sc=False),
    )(dst_rows, init)


KG = 4
NG = RPT // KG


def _make_agg_body(stage_table):
    def _agg_body(g_hbm, src_hbm, dst_hbm, zeros_hbm, out_hbm,
                  src_v, dst_v, vals_v, stage_v, acc_sh, g_sh,
                  gs_a, gs_b, ss_a, ss_b):
        c = lax.axis_index("c")
        s = lax.axis_index("s")
        row0 = (c * NS + s) * RPT
        pltpu.sync_copy(src_hbm.at[pl.ds(row0, RPT)], src_v)
        pltpu.sync_copy(dst_hbm.at[pl.ds(row0, RPT)], dst_v)
        if stage_table:
            pltpu.sync_copy(g_hbm.at[pl.ds(s * RT, RT)], stage_v)
            pltpu.sync_copy(stage_v, g_sh.at[pl.ds(s * RT, RT)])
        pltpu.sync_copy(zeros_hbm.at[pl.ds(s * RT, RT)], stage_v)
        pltpu.sync_copy(stage_v, acc_sh.at[pl.ds(s * RT, RT)])
        plsc.subcore_barrier()

        gtab = g_sh if stage_table else g_hbm
        gsems = (gs_a, gs_b)
        ssems = (ss_a, ss_b)
        last = RPT - 1

        def gath(phase, b, j):
            pltpu.async_copy(gtab.at[src_v.at[j]], vals_v.at[phase, b],
                             gsems[phase])

        def drain_g(phase, b):
            pltpu.make_async_copy(g_hbm.at[pl.ds(0, CHUNK)],
                                  vals_v.at[phase, b], gsems[phase]).wait()

        def scat(phase, b, j):
            pltpu.async_copy(vals_v.at[phase, b], acc_sh.at[dst_v.at[j]],
                             ssems[phase], add=True)

        def drain_s(phase, b):
            pltpu.make_async_copy(g_hbm.at[pl.ds(0, CHUNK)],
                                  vals_v.at[phase, b], ssems[phase]).wait()

        for b in range(KG):
            gath(0, b, b)
        for b in range(KG):
            gath(1, b, KG + b)

        def body(i, carry):
            base_a = (2 * i) * KG
            base_b = base_a + KG
            for b in range(KG):
                drain_g(0, b)
            for b in range(KG):
                scat(0, b, base_a + b)
            for b in range(KG):
                drain_g(1, b)
            for b in range(KG):
                scat(1, b, base_b + b)
            for b in range(KG):
                drain_s(0, b)
            for b in range(KG):
                gath(0, b, jnp.minimum(base_a + 2 * KG + b, last))
            for b in range(KG):
                drain_s(1, b)
            for b in range(KG):
                gath(1, b, jnp.minimum(base_b + 2 * KG + b, last))
            return carry

        lax.fori_loop(0, NG // 2, body, 0)
        for phase in (0, 1):
            for b in range(KG):
                drain_g(phase, b)
        plsc.subcore_barrier()
        pltpu.sync_copy(acc_sh.at[pl.ds(s * RT, RT)], stage_v)
        pltpu.sync_copy(stage_v, out_hbm.at[c, pl.ds(s * RT, RT)])

    return _agg_body


def _seg_sum_sc(g, src_rows, dst_rows, feat, zeros, stage_table):
    g = jnp.pad(g, ((0, NPAD - N), (0, 0)))
    gsh_rows = NPAD if stage_table else 8
    return pl.kernel(
        _make_agg_body(stage_table),
        out_type=jax.ShapeDtypeStruct((NC, NPAD, feat), jnp.float32),
        mesh=_MESH,
        scratch_types=[
            pltpu.VMEM((RPT, CHUNK), jnp.int32),
            pltpu.VMEM((RPT, CHUNK), jnp.int32),
            pltpu.VMEM((2, KG, CHUNK, feat), jnp.float32),
            pltpu.VMEM((RT, feat), jnp.float32),
            pltpu.VMEM_SHARED((NPAD, feat), jnp.float32),
            pltpu.VMEM_SHARED((gsh_rows, feat), jnp.float32),
            pltpu.SemaphoreType.DMA,
            pltpu.SemaphoreType.DMA,
            pltpu.SemaphoreType.DMA,
            pltpu.SemaphoreType.DMA,
        ],
        compiler_params=pltpu.CompilerParams(use_tc_tiling_on_sc=False),
    )(g, src_rows, dst_rows, zeros)


def _a0_body(x_ref, w0pt_ref, w0t_ref, dinv_ref, tanoff_ref, g_ref):
    xx = x_ref[...]
    off = jnp.clip(jnp.dot(xx, w0pt_ref[...], preferred_element_type=jnp.float32),
                   LO, HI)
    tanoff_ref[...] = jnp.tan(off)
    g_ref[...] = dinv_ref[...] * jnp.dot(xx, w0t_ref[...],
                                         preferred_element_type=jnp.float32)


def _a1_body(tanoff_ref, a0_ref, a1_ref, g_ref, dinv_ref, b_ref, wt_ref,
             tanoff_o, g_o):
    dinv = dinv_ref[...]
    y = (tanoff_ref[...]
         + dinv * (a0_ref[...] + a1_ref[...] + g_ref[...]) + b_ref[...])
    h = jnp.arctan2(y, 1.0)
    tanoff_o[...] = jnp.clip(y, TLO, THI)
    g_o[...] = dinv * jnp.dot(h, wt_ref[...], preferred_element_type=jnp.float32)


def _a2_body(tanoff_ref, a0_ref, a1_ref, g_ref, dinv_ref, b_ref, wpt_ref, wt_ref,
             tanoff_o, g_o):
    dinv = dinv_ref[...]
    h = jnp.arctan2(tanoff_ref[...]
                    + dinv * (a0_ref[...] + a1_ref[...] + g_ref[...])
                    + b_ref[...], 1.0)
    off = jnp.dot(h, wpt_ref[...], preferred_element_type=jnp.float32)
    tanoff_o[...] = jnp.tan(jnp.clip(off, LO, HI))
    g_o[...] = dinv * jnp.dot(h, wt_ref[...], preferred_element_type=jnp.float32)


def _b2_body(tanoff_ref, a0_ref, a1_ref, g_ref, dinv_ref, b_ref, out_ref):
    h = jnp.arctan2(tanoff_ref[...]
                    + dinv_ref[...] * (a0_ref[...] + a1_ref[...] + g_ref[...])
                    + b_ref[...], 1.0)
    m = jnp.max(h, axis=1, keepdims=True)
    lse = m + jnp.log(jnp.sum(jnp.exp(h - m), axis=1, keepdims=True))
    out_ref[...] = h - lse


def _sds(shape):
    return jax.ShapeDtypeStruct(shape, jnp.float32)


def _proj_simplex(v, radius=1.0):
    n_feat = v.shape[1]
    u = jnp.sort(v, axis=1)
    cssv = jnp.cumsum(u, axis=1) - radius
    ind = jnp.arange(1, n_feat + 1)
    cond = u - cssv / ind.astype(v.dtype) > 0
    rho = jnp.max(jnp.where(cond, ind, 0), axis=1)
    theta = jnp.take_along_axis(cssv, (rho - 1)[:, None], axis=1)[:, 0]
    theta = theta / rho.astype(v.dtype)
    return jax.nn.relu(v - theta[:, None])


def kernel(x, edge_index, W_rep0, W_rep2, W0, b0, W1, b1, W2, b2):
    f32 = jnp.float32
    src = edge_index[0]
    dst = edge_index[1]
    pad = EPAD - E
    src_rows = jnp.concatenate(
        [src, jnp.zeros((pad,), jnp.int32)]).reshape(ROWS, CHUNK)
    dst_rows = jnp.concatenate(
        [dst, jnp.full((pad,), JUNK, jnp.int32)]).reshape(ROWS, CHUNK)

    deg_init = jnp.concatenate(
        [jnp.ones((NPAD,), f32), jnp.zeros((NPAD,), f32)])
    degp = _deg_sc(dst_rows, deg_init)
    deg = degp[:N] + degp[NPAD:NPAD + N]
    dinv = (deg ** -0.5).reshape(N, 1)

    W0p = _proj_simplex(W_rep0)
    W2p = _proj_simplex(W_rep2)

    zeros16 = jnp.zeros((NPAD, 16), f32)
    zeros24 = jnp.zeros((NPAD, 24), f32)

    tanoff0, g0 = pl.pallas_call(
        _a0_body, out_shape=[_sds((N, 16)), _sds((N, 16))],
    )(x, W0p.T, W0.T, dinv)

    agg0 = _seg_sum_sc(g0, src_rows, dst_rows, 16, zeros16, True)

    tanoff1, g1 = pl.pallas_call(
        _a1_body, out_shape=[_sds((N, 16)), _sds((N, 16))],
    )(tanoff0, agg0[0, :N], agg0[1, :N], g0, dinv, b0.reshape(1, 16), W1.T)

    agg1 = _seg_sum_sc(g1, src_rows, dst_rows, 16, zeros16, True)

    tanoff2, g2 = pl.pallas_call(
        _a2_body, out_shape=[_sds((N, 40)), _sds((N, 40))],
    )(tanoff1, agg1[0, :N], agg1[1, :N], g1, dinv, b1.reshape(1, 16),
      W2p.T, W2.T)

    agg2a = _seg_sum_sc(g2[:, :16], src_rows, dst_rows, 16, zeros16, True)
    agg2b = _seg_sum_sc(g2[:, 16:], src_rows, dst_rows, 24, zeros24, True)
    a20 = jnp.concatenate([agg2a[0, :N], agg2b[0, :N]], axis=1)
    a21 = jnp.concatenate([agg2a[1, :N], agg2b[1, :N]], axis=1)

    out = pl.pallas_call(
        _b2_body, out_shape=_sds((N, 40)),
    )(tanoff2, a20, a21, g2, dinv, b2.reshape(1, 40))
    return out


# speedup vs baseline: 32.2580x; 1.0017x over previous
import math

import jax
import jax.numpy as jnp
from jax import lax
from jax.experimental import pallas as pl
from jax.experimental.pallas import tpu as pltpu
from jax.experimental.pallas import tpu_sc as plsc

N = 10000
E = 320000
LO = -math.pi / 2 + 0.01
HI = math.pi / 2 - 0.01
THI = math.tan(HI)
TLO = -THI

NC, NS = 2, 16
NW = NC * NS
CHUNK = 128
RPT = 80
ROWS = RPT * NW
EPAD = ROWS * CHUNK
NPAD = 10112
JUNK = N
RT = NPAD // NS

_MESH = plsc.VectorSubcoreMesh(core_axis_name="c", subcore_axis_name="s")
_SC_PARAMS = pltpu.CompilerParams(use_tc_tiling_on_sc=False)


def _deg_body(dst_hbm, init_hbm, out_hbm, dst_v, ones_v, stage_v, acc_sh):
    c = lax.axis_index("c")
    s = lax.axis_index("s")
    row0 = (c * NS + s) * RPT
    pltpu.sync_copy(dst_hbm.at[pl.ds(row0, RPT)], dst_v)
    pltpu.sync_copy(init_hbm.at[0, pl.ds(0, CHUNK)], ones_v)
    pltpu.sync_copy(init_hbm.at[c, pl.ds(s * RT, RT)], stage_v)
    pltpu.sync_copy(stage_v, acc_sh.at[pl.ds(s * RT, RT)])
    plsc.subcore_barrier()

    def step(j, carry):
        pltpu.sync_copy(ones_v, acc_sh.at[dst_v.at[j]], add=True)
        return carry

    lax.fori_loop(0, RPT, step, 0)
    plsc.subcore_barrier()
    pltpu.sync_copy(acc_sh.at[pl.ds(s * RT, RT)], stage_v)
    pltpu.sync_copy(stage_v, out_hbm.at[c, pl.ds(s * RT, RT)])


def _deg_sc(dst_rows, init):
    return pl.kernel(
        _deg_body,
        out_type=jax.ShapeDtypeStruct((NC, NPAD, 8), jnp.float32),
        mesh=_MESH,
        scratch_types=[
            pltpu.VMEM((RPT, CHUNK), jnp.int32),
            pltpu.VMEM((CHUNK, 8), jnp.float32),
            pltpu.VMEM((RT, 8), jnp.float32),
            pltpu.VMEM_SHARED((NPAD, 8), jnp.float32),
        ],
        compiler_params=_SC_PARAMS,
    )(dst_rows, init)


KG = 4
NG = RPT // KG


def _agg_body(g_hbm, src_hbm, dst_hbm, zeros_hbm, out_hbm,
              src_v, dst_v, vals_v, stage_v, acc_sh, g_sh,
              gs_a, gs_b, ss_a, ss_b):
    c = lax.axis_index("c")
    s = lax.axis_index("s")
    row0 = (c * NS + s) * RPT
    pltpu.sync_copy(src_hbm.at[pl.ds(row0, RPT)], src_v)
    pltpu.sync_copy(dst_hbm.at[pl.ds(row0, RPT)], dst_v)
    pltpu.sync_copy(g_hbm.at[pl.ds(s * RT, RT)], stage_v)
    pltpu.sync_copy(stage_v, g_sh.at[pl.ds(s * RT, RT)])
    pltpu.sync_copy(zeros_hbm.at[pl.ds(s * RT, RT)], stage_v)
    pltpu.sync_copy(stage_v, acc_sh.at[pl.ds(s * RT, RT)])
    plsc.subcore_barrier()

    gsems = (gs_a, gs_b)
    ssems = (ss_a, ss_b)
    last = RPT - 1

    def gath(phase, b, j):
        pltpu.async_copy(g_sh.at[src_v.at[j]], vals_v.at[phase, b],
                         gsems[phase])

    def drain_g(phase, b):
        pltpu.make_async_copy(g_hbm.at[pl.ds(0, CHUNK)],
                              vals_v.at[phase, b], gsems[phase]).wait()

    def scat(phase, b, j):
        pltpu.async_copy(vals_v.at[phase, b], acc_sh.at[dst_v.at[j]],
                         ssems[phase], add=True)

    def drain_s(phase, b):
        pltpu.make_async_copy(g_hbm.at[pl.ds(0, CHUNK)],
                              vals_v.at[phase, b], ssems[phase]).wait()

    for b in range(KG):
        gath(0, b, b)
    for b in range(KG):
        gath(1, b, KG + b)

    def body(i, carry):
        base_a = (2 * i) * KG
        base_b = base_a + KG
        for b in range(KG):
            drain_g(0, b)
        for b in range(KG):
            scat(0, b, base_a + b)
        for b in range(KG):
            drain_g(1, b)
        for b in range(KG):
            scat(1, b, base_b + b)
        for b in range(KG):
            drain_s(0, b)
        for b in range(KG):
            gath(0, b, jnp.minimum(base_a + 2 * KG + b, last))
        for b in range(KG):
            drain_s(1, b)
        for b in range(KG):
            gath(1, b, jnp.minimum(base_b + 2 * KG + b, last))
        return carry

    lax.fori_loop(0, NG // 2, body, 0)
    for phase in (0, 1):
        for b in range(KG):
            drain_g(phase, b)
    plsc.subcore_barrier()
    pltpu.sync_copy(acc_sh.at[pl.ds(s * RT, RT)], stage_v)
    pltpu.sync_copy(stage_v, out_hbm.at[c, pl.ds(s * RT, RT)])


def _seg_sum_sc(g_padded, src_rows, dst_rows, feat, zeros):
    return pl.kernel(
        _agg_body,
        out_type=jax.ShapeDtypeStruct((NC, NPAD, feat), jnp.float32),
        mesh=_MESH,
        scratch_types=[
            pltpu.VMEM((RPT, CHUNK), jnp.int32),
            pltpu.VMEM((RPT, CHUNK), jnp.int32),
            pltpu.VMEM((2, KG, CHUNK, feat), jnp.float32),
            pltpu.VMEM((RT, feat), jnp.float32),
            pltpu.VMEM_SHARED((NPAD, feat), jnp.float32),
            pltpu.VMEM_SHARED((NPAD, feat), jnp.float32),
            pltpu.SemaphoreType.DMA,
            pltpu.SemaphoreType.DMA,
            pltpu.SemaphoreType.DMA,
            pltpu.SemaphoreType.DMA,
        ],
        compiler_params=_SC_PARAMS,
    )(g_padded, src_rows, dst_rows, zeros)


def _a0_body(x_ref, w0pt_ref, w0t_ref, degp_ref, tanoff_ref, g_ref, dinv_ref):
    xx = x_ref[...]
    dp = degp_ref[...]
    deg = dp[0, :N, :1] + dp[1, :N, :1]
    dinv = lax.rsqrt(deg)
    dinv_ref[...] = dinv
    off = jnp.clip(jnp.dot(xx, w0pt_ref[...], preferred_element_type=jnp.float32),
                   LO, HI)
    tanoff_ref[...] = jnp.tan(off)
    g = dinv * jnp.dot(xx, w0t_ref[...], preferred_element_type=jnp.float32)
    g_ref[...] = jnp.concatenate(
        [g, jnp.zeros((NPAD - N, 16), jnp.float32)], axis=0)


def _a1_body(tanoff_ref, agg_ref, g_ref, dinv_ref, b_ref, wt_ref,
             tanoff_o, g_o):
    dinv = dinv_ref[...]
    a = agg_ref[...]
    y = (tanoff_ref[...]
         + dinv * (a[0, :N] + a[1, :N] + g_ref[...][:N]) + b_ref[...])
    h = jnp.arctan2(y, 1.0)
    tanoff_o[...] = jnp.clip(y, TLO, THI)
    g = dinv * jnp.dot(h, wt_ref[...], preferred_element_type=jnp.float32)
    g_o[...] = jnp.concatenate(
        [g, jnp.zeros((NPAD - N, 16), jnp.float32)], axis=0)


def _a2_body(tanoff_ref, agg_ref, g_ref, dinv_ref, b_ref, wpt_ref, wt_ref,
             tanoff_o, ga_o, gb_o):
    dinv = dinv_ref[...]
    a = agg_ref[...]
    h = jnp.arctan2(tanoff_ref[...]
                    + dinv * (a[0, :N] + a[1, :N] + g_ref[...][:N])
                    + b_ref[...], 1.0)
    off = jnp.dot(h, wpt_ref[...], preferred_element_type=jnp.float32)
    tanoff_o[...] = jnp.tan(jnp.clip(off, LO, HI))
    g = dinv * jnp.dot(h, wt_ref[...], preferred_element_type=jnp.float32)
    ga_o[...] = jnp.concatenate(
        [g[:, :16], jnp.zeros((NPAD - N, 16), jnp.float32)], axis=0)
    gb_o[...] = jnp.concatenate(
        [g[:, 16:], jnp.zeros((NPAD - N, 24), jnp.float32)], axis=0)


def _b2_body(tanoff_ref, agga_ref, aggb_ref, ga_ref, gb_ref, dinv_ref, b_ref,
             out_ref):
    dinv = dinv_ref[...]
    aa = agga_ref[...]
    ab = aggb_ref[...]
    sa = aa[0, :N] + aa[1, :N] + ga_ref[...][:N]
    sb = ab[0, :N] + ab[1, :N] + gb_ref[...][:N]
    h = jnp.arctan2(tanoff_ref[...]
                    + dinv * jnp.concatenate([sa, sb], axis=1)
                    + b_ref[...], 1.0)
    m = jnp.max(h, axis=1, keepdims=True)
    lse = m + jnp.log(jnp.sum(jnp.exp(h - m), axis=1, keepdims=True))
    out_ref[...] = h - lse


def _sds(shape):
    return jax.ShapeDtypeStruct(shape, jnp.float32)


def _proj_simplex(v, radius=1.0):
    n_feat = v.shape[1]
    u = jnp.sort(v, axis=1)
    cssv = jnp.cumsum(u, axis=1) - radius
    ind = jnp.arange(1, n_feat + 1)
    cond = u - cssv / ind.astype(v.dtype) > 0
    rho = jnp.max(jnp.where(cond, ind, 0), axis=1)
    theta = jnp.take_along_axis(cssv, (rho - 1)[:, None], axis=1)[:, 0]
    theta = theta / rho.astype(v.dtype)
    return jax.nn.relu(v - theta[:, None])


def kernel(x, edge_index, W_rep0, W_rep2, W0, b0, W1, b1, W2, b2):
    f32 = jnp.float32
    src = edge_index[0]
    dst = edge_index[1]
    pad = EPAD - E
    src_rows = jnp.concatenate(
        [src, jnp.zeros((pad,), jnp.int32)]).reshape(ROWS, CHUNK)
    dst_rows = jnp.concatenate(
        [dst, jnp.full((pad,), JUNK, jnp.int32)]).reshape(ROWS, CHUNK)

    deg_init = jnp.concatenate(
        [jnp.ones((1, NPAD, 8), f32), jnp.zeros((1, NPAD, 8), f32)])
    degp = _deg_sc(dst_rows, deg_init)

    W0p = _proj_simplex(W_rep0)
    W2p = _proj_simplex(W_rep2)

    zeros16 = jnp.zeros((NPAD, 16), f32)
    zeros24 = jnp.zeros((NPAD, 24), f32)

    tanoff0, g0, dinv = pl.pallas_call(
        _a0_body, out_shape=[_sds((N, 16)), _sds((NPAD, 16)), _sds((N, 1))],
    )(x, W0p.T, W0.T, degp)

    agg0 = _seg_sum_sc(g0, src_rows, dst_rows, 16, zeros16)

    tanoff1, g1 = pl.pallas_call(
        _a1_body, out_shape=[_sds((N, 16)), _sds((NPAD, 16))],
    )(tanoff0, agg0, g0, dinv, b0.reshape(1, 16), W1.T)

    agg1 = _seg_sum_sc(g1, src_rows, dst_rows, 16, zeros16)

    tanoff2, g2a, g2b = pl.pallas_call(
        _a2_body, out_shape=[_sds((N, 40)), _sds((NPAD, 16)), _sds((NPAD, 24))],
    )(tanoff1, agg1, g1, dinv, b1.reshape(1, 16), W2p.T, W2.T)

    agg2a = _seg_sum_sc(g2a, src_rows, dst_rows, 16, zeros16)
    agg2b = _seg_sum_sc(g2b, src_rows, dst_rows, 24, zeros24)

    out = pl.pallas_call(
        _b2_body, out_shape=_sds((N, 40)),
    )(tanoff2, agg2a, agg2b, g2a, g2b, dinv, b2.reshape(1, 40))
    return out
